# baseline (speedup 1.0000x reference)
import functools

import jax
import jax.numpy as jnp
from jax import lax
from jax.experimental import pallas as pl
from jax.experimental.pallas import tpu as pltpu
from jax.experimental.pallas import tpu_sc as plsc

N_KN = 10000
N_EMP = 50000
EMB = 128

NC = 2
NS = 16
NW = NC * NS
NT = 4
CW = EMB // NT

TRASH = N_EMP
R_ACC = 50176
RPS = R_ACC // NS
ZCH = 112

NGB0, NGB1 = 64, 16
NGC0, NGC1 = 160, 80
NBUF = 4

_mesh = lambda: plsc.VectorSubcoreMesh(core_axis_name="c", subcore_axis_name="s")
_sc_params = lambda: pltpu.CompilerParams(use_tc_tiling_on_sc=False,
                                          needs_layout_passes=False)


def _prep_chunks(a, ng0, ng1, fill):
    n0, n1 = NS * ng0 * 128, NS * ng1 * 128
    pad = n0 + n1 - a.shape[0]
    if fill == TRASH:
        padv = TRASH + (jnp.arange(pad, dtype=a.dtype) % (R_ACC - N_EMP))
    else:
        padv = jnp.full((pad,), fill, a.dtype)
    a = jnp.concatenate([a, padv])
    ngm = max(ng0, ng1)
    p0 = a[:n0].reshape(NS, ng0, 128)
    p1 = a[n0:].reshape(NS, ng1, 128)
    p0 = jnp.pad(p0, ((0, 0), (0, ngm - ng0), (0, 0)), constant_values=fill)
    p1 = jnp.pad(p1, ((0, 0), (0, ngm - ng1), (0, 0)), constant_values=fill)
    return jnp.concatenate([p0, p1], axis=0)


def _zero_accum(acc, zbuf, s, zsem):
    @pl.loop(0, RPS // ZCH)
    def _(z):
        pltpu.async_copy(zbuf, acc.at[pl.ds(s * RPS + z * ZCH, ZCH)], zsem)

    @pl.loop(0, RPS // ZCH)
    def _(z):
        pltpu.make_async_copy(zbuf, acc.at[pl.ds(0, ZCH)], zsem).wait()


def _deg_call(dstb, dstc, ones_hbm):
    @functools.partial(
        pl.kernel,
        mesh=_mesh(),
        compiler_params=_sc_params(),
        out_type=jax.ShapeDtypeStruct((4, R_ACC, 16), jnp.float32),
        scratch_types=[
            pltpu.VMEM((NGB0, 128), jnp.int32),
            pltpu.VMEM((NGC0, 128), jnp.int32),
            pltpu.VMEM((128, 16), jnp.float32),
            pltpu.VMEM((ZCH, 16), jnp.float32),
            pltpu.VMEM_SHARED((R_ACC, 16), jnp.float32),
            pltpu.SemaphoreType.DMA,
            pltpu.SemaphoreType.DMA,
        ],
    )
    def k(dstb_hbm, dstc_hbm, ones_h, out_hbm, dbv, dcv, ones_v, zbuf, dacc,
          lsem, ssem):
        c = lax.axis_index("c")
        s = lax.axis_index("s")
        row = c * NS + s
        pltpu.sync_copy(ones_h, ones_v)

        @pl.loop(0, ZCH)
        def _(r):
            zbuf[r, pl.ds(0, 16)] = jnp.zeros((16,), jnp.float32)

        for set_i, (dh, dv, ng0, ng1) in enumerate(
                [(dstb_hbm, dbv, NGB0, NGB1), (dstc_hbm, dcv, NGC0, NGC1)]):
            ngc = jnp.where(c == 0, ng0, ng1)
            pltpu.sync_copy(dh.at[row], dv)
            _zero_accum(dacc, zbuf, s, lsem)
            plsc.subcore_barrier()

            @pl.loop(0, ngc)
            def _(g):
                pltpu.async_copy(ones_v, dacc.at[dv.at[g]], ssem, add=True)

                @pl.when(g >= 8)
                def _():
                    pltpu.make_async_copy(ones_h, ones_v, ssem).wait()

            @pl.loop(0, jnp.minimum(ngc, 8))
            def _(g):
                pltpu.make_async_copy(ones_h, ones_v, ssem).wait()

            plsc.subcore_barrier()
            pltpu.sync_copy(dacc.at[pl.ds(s * RPS, RPS)],
                            out_hbm.at[set_i * 2 + c, pl.ds(s * RPS, RPS)])
            plsc.subcore_barrier()

    return k(dstb, dstc, ones_hbm)


def _segsum_call(tables, srcc, dstc, ewc, ng0, ng1):
    weighted = ewc is not None
    ngm = max(ng0, ng1)

    scratch = (
        [pltpu.VMEM((NBUF, 128), jnp.int32) for _ in range(2)]
        + [pltpu.VMEM((NBUF, 128), jnp.int32) for _ in range(2)]
        + ([pltpu.VMEM((NBUF * 128,), jnp.float32) for _ in range(2)]
           if weighted else [])
        + [pltpu.VMEM((128, CW), jnp.float32) for _ in range(NBUF)]
        + [pltpu.VMEM((ZCH, CW), jnp.float32),
           pltpu.VMEM_SHARED((R_ACC, CW), jnp.float32)]
        + [pltpu.SemaphoreType.DMA for _ in range(2 * NBUF + 3)]
    )

    @functools.partial(
        pl.kernel,
        mesh=_mesh(),
        compiler_params=_sc_params(),
        out_type=jax.ShapeDtypeStruct((NC, NT, R_ACC, CW), jnp.float32),
        scratch_types=scratch,
    )
    def k(tab, src_hbm, dst_hbm, *rest):
        if weighted:
            (ew_hbm, out_hbm, sv0, sv1, dv0, dv1, ev0, ev1, *rest2) = rest
            evs = (ev0, ev1)
        else:
            (out_hbm, sv0, sv1, dv0, dv1, *rest2) = rest
        svs = (sv0, sv1)
        dvs = (dv0, dv1)
        bufs = rest2[:NBUF]
        zbuf = rest2[NBUF]
        acc = rest2[NBUF + 1]
        gsems = rest2[NBUF + 2:2 * NBUF + 2]
        ssems = rest2[2 * NBUF + 2:3 * NBUF + 2]
        isems = rest2[3 * NBUF + 2:3 * NBUF + 4]
        zsem = rest2[3 * NBUF + 4]
        c = lax.axis_index("c")
        s = lax.axis_index("s")
        wid = c * NS + s
        ng = jnp.where(c == 0, ng0, ng1)
        nch = ng // NBUF

        @pl.loop(0, ZCH)
        def _(r):
            zbuf[r, pl.ds(0, 16)] = jnp.zeros((16,), jnp.float32)
            zbuf[r, pl.ds(16, 16)] = jnp.zeros((16,), jnp.float32)

        def fetch_idx_sync(ci, p):
            pltpu.sync_copy(src_hbm.at[wid, pl.ds(ci * NBUF, NBUF)], svs[p])
            pltpu.sync_copy(dst_hbm.at[wid, pl.ds(ci * NBUF, NBUF)], dvs[p])
            if weighted:
                pltpu.sync_copy(
                    ew_hbm.at[wid, pl.ds(ci * NBUF * 128, NBUF * 128)],
                    evs[p])

        def fetch_idx_async(ci, p):
            pltpu.async_copy(src_hbm.at[wid, pl.ds(ci * NBUF, NBUF)],
                             svs[p], isems[p])
            pltpu.async_copy(dst_hbm.at[wid, pl.ds(ci * NBUF, NBUF)],
                             dvs[p], isems[p])
            if weighted:
                pltpu.async_copy(
                    ew_hbm.at[wid, pl.ds(ci * NBUF * 128, NBUF * 128)],
                    evs[p], isems[p])

        def wait_idx(p):
            pltpu.make_async_copy(src_hbm.at[0, pl.ds(0, NBUF)], svs[p],
                                  isems[p]).wait()
            pltpu.make_async_copy(dst_hbm.at[0, pl.ds(0, NBUF)], dvs[p],
                                  isems[p]).wait()
            if weighted:
                pltpu.make_async_copy(
                    ew_hbm.at[0, pl.ds(0, NBUF * 128)], evs[p],
                    isems[p]).wait()

        def gather(t, p, kk, b):
            pltpu.async_copy(tab.at[t].at[svs[p].at[kk]], bufs[b], gsems[b])

        def wait_gather(b):
            pltpu.make_async_copy(tab.at[0].at[pl.ds(0, 128)], bufs[b],
                                  gsems[b]).wait()

        def drain_scatter(b):
            pltpu.make_async_copy(tab.at[0].at[pl.ds(0, 128)], bufs[b],
                                  ssems[b]).wait()

        @pl.loop(0, NT)
        def _(t):
            _zero_accum(acc, zbuf, s, zsem)
            plsc.subcore_barrier()

            fetch_idx_sync(0, 0)
            gather(t, 0, 0, 0)
            gather(t, 0, 1, 1)

            @pl.loop(0, nch, step=2)
            def _(ci0):
                for p in range(2):
                    ci = ci0 + p
                    for kk in range(NBUF):
                        h = ci * NBUF + kk
                        b = kk
                        bn = (kk + 2) % NBUF
                        wait_gather(b)
                        if weighted:
                            rows = bufs[b]
                            ewv = evs[p]

                            @pl.loop(0, 128, step=8)
                            def _(e0):
                                for dd in range(8):
                                    e = e0 + dd
                                    w = plsc.load_gather(
                                        ewv,
                                        [jnp.full((16,), kk * 128,
                                                  jnp.int32) + e])
                                    rows[e, pl.ds(0, 16)] = (
                                        rows[e, pl.ds(0, 16)] * w)
                                    rows[e, pl.ds(16, 16)] = (
                                        rows[e, pl.ds(16, 16)] * w)

                        pltpu.async_copy(bufs[b], acc.at[dvs[p].at[kk]],
                                         ssems[b], add=True)

                        @pl.when(h >= 2)
                        def _():
                            drain_scatter(bn)

                        if kk == 1:
                            @pl.when(ci + 1 < nch)
                            def _():
                                fetch_idx_async(ci + 1, 1 - p)
                        if kk == 2:
                            @pl.when(ci + 1 < nch)
                            def _():
                                wait_idx(1 - p)

                        pn = p if kk < 2 else 1 - p
                        kn = (kk + 2) % NBUF

                        @pl.when(h + 2 < ng)
                        def _():
                            gather(t, pn, kn, bn)

            drain_scatter(2)
            drain_scatter(3)
            plsc.subcore_barrier()
            pltpu.sync_copy(acc.at[pl.ds(s * RPS, RPS)],
                            out_hbm.at[c, t, pl.ds(s * RPS, RPS)])
            plsc.subcore_barrier()

    args = (tables, srcc, dstc)
    if weighted:
        args = args + (ewc,)
    return k(*args)


def _dotT(x, w):
    return lax.dot_general(x.astype(jnp.bfloat16), w.astype(jnp.bfloat16),
                           (((1,), (1,)), ((), ())),
                           preferred_element_type=jnp.float32)


def _full(a):
    return pl.BlockSpec(a.shape, lambda i: (0,) * a.ndim)


def _stage1(ii, kw, w4_W, w4_b):
    br = 2000

    def body(ii_r, kw_r, w_r, b_r, o_r):
        y = (_dotT(ii_r[...], w_r[:, :EMB]) + _dotT(kw_r[...], w_r[:, EMB:])
             + b_r[...])
        for t in range(NT):
            o_r[t] = y[:, t * CW:(t + 1) * CW]

    return pl.pallas_call(
        body,
        grid=(N_KN // br,),
        in_specs=[
            pl.BlockSpec((br, EMB), lambda i: (i, 0)),
            pl.BlockSpec((br, EMB), lambda i: (i, 0)),
            _full(w4_W),
            _full(w4_b),
        ],
        out_specs=pl.BlockSpec((NT, br, CW), lambda i: (0, i, 0)),
        out_shape=jax.ShapeDtypeStruct((NT, N_KN, CW), jnp.float32),
    )(ii, kw, w4_W, w4_b)


def _strip_mats(hp_r, deg, Wn0, Wn1):
    recip = 1.0 / jnp.maximum(deg, 1.0)
    hn0 = None
    hn1 = None
    for t in range(NT):
        st = (hp_r[0, t] + hp_r[1, t]) * recip
        c0 = _dotT(st, Wn0[:, t * CW:(t + 1) * CW])
        c1 = _dotT(st, Wn1[:, t * CW:(t + 1) * CW])
        hn0 = c0 if hn0 is None else hn0 + c0
        hn1 = c1 if hn1 is None else hn1 + c1
    return hn0, hn1


def _stage3(hnbp, degp, e_emb, ek_Wself, ek_Wneigh, ek_b, w2_W, w2_b,
            w1_W, w1_b):
    br = 2000

    def body(hp_r, dg_r, e_r, ws_r, wn_r, b_r, w2_r, w2b_r, w1_r, w1b_r,
             o_hi, o_sc):
        degb = dg_r[0, :, 0:1] + dg_r[1, :, 0:1]
        hn0, hn1 = _strip_mats(hp_r, degb, wn_r[0], wn_r[1])
        e = e_r[...]
        d1 = jnp.maximum(_dotT(e, ws_r[0]) + hn0 + b_r[0], 0.0)
        hi = _dotT(d1, ws_r[1]) + hn1 + b_r[1]
        hi2 = _dotT(hi, w2_r[...]) + w2b_r[...]
        o_hi[...] = hi2
        sc = (_dotT(hi2, w1_r[:, :EMB]) + _dotT(e, w1_r[:, EMB:])
              + w1b_r[...])
        for t in range(NT):
            o_sc[t] = sc[:, t * CW:(t + 1) * CW]

    return pl.pallas_call(
        body,
        grid=(N_EMP // br,),
        in_specs=[
            pl.BlockSpec((NC, NT, br, CW), lambda i: (0, 0, i, 0)),
            pl.BlockSpec((4, br, 16), lambda i: (0, i, 0)),
            pl.BlockSpec((br, EMB), lambda i: (i, 0)),
            _full(ek_Wself), _full(ek_Wneigh), _full(ek_b),
            _full(w2_W), _full(w2_b), _full(w1_W), _full(w1_b),
        ],
        out_specs=[
            pl.BlockSpec((br, EMB), lambda i: (i, 0)),
            pl.BlockSpec((NT, br, CW), lambda i: (0, i, 0)),
        ],
        out_shape=[
            jax.ShapeDtypeStruct((N_EMP, EMB), jnp.float32),
            jax.ShapeDtypeStruct((NT, N_EMP, CW), jnp.float32),
        ],
    )(hnbp, degp, e_emb, ek_Wself, ek_Wneigh, ek_b, w2_W, w2_b, w1_W, w1_b)


def _stage5(hncp, degp, e_emb, h_iI, ee_Wself, ee_Wneigh, ee_b, w3_W, w3_b,
            comb_W, comb_b):
    br = 2000

    def body(hp_r, dg_r, e_r, hi_r, ws_r, wn_r, b_r, w3_r, w3b_r, cb_r,
             cbb_r, o_r):
        degc = dg_r[2, :, 0:1] + dg_r[3, :, 0:1]
        hn0, hn1 = _strip_mats(hp_r, degc, wn_r[0], wn_r[1])
        e = e_r[...]
        d2 = jnp.maximum(_dotT(e, ws_r[0]) + hn0 + b_r[0], 0.0)
        hs = _dotT(d2, ws_r[1]) + hn1 + b_r[1]
        hs2 = _dotT(hs, w3_r[...]) + w3b_r[...]
        h = (_dotT(hi_r[...], cb_r[:, :EMB]) + _dotT(hs2, cb_r[:, EMB:])
             + cbb_r[...])
        o_r[...] = jnp.where(h >= 0, h, 0.2 * h)

    return pl.pallas_call(
        body,
        grid=(N_EMP // br,),
        in_specs=[
            pl.BlockSpec((NC, NT, br, CW), lambda i: (0, 0, i, 0)),
            pl.BlockSpec((4, br, 16), lambda i: (0, i, 0)),
            pl.BlockSpec((br, EMB), lambda i: (i, 0)),
            pl.BlockSpec((br, EMB), lambda i: (i, 0)),
            _full(ee_Wself), _full(ee_Wneigh), _full(ee_b),
            _full(w3_W), _full(w3_b), _full(comb_W), _full(comb_b),
        ],
        out_specs=pl.BlockSpec((br, EMB), lambda i: (i, 0)),
        out_shape=jax.ShapeDtypeStruct((N_EMP, EMB), jnp.float32),
    )(hncp, degp, e_emb, h_iI, ee_Wself, ee_Wneigh, ee_b, w3_W, w3_b,
      comb_W, comb_b)


def kernel(ii, e_emb, cf_ew, belong_edge_index, collab_edge_index, k_emb_w,
           w4_W, w4_b, w1_W, w1_b, w2_W, w2_b, w3_W, w3_b, comb_W, comb_b,
           ek_Wself, ek_Wneigh, ek_b, ee_Wself, ee_Wneigh, ee_b):
    b_src = _prep_chunks(belong_edge_index[0], NGB0, NGB1, 0)
    b_dst = _prep_chunks(belong_edge_index[1], NGB0, NGB1, TRASH)
    c_src = _prep_chunks(collab_edge_index[0], NGC0, NGC1, 0)
    c_dst = _prep_chunks(collab_edge_index[1], NGC0, NGC1, TRASH)
    ew = _prep_chunks(cf_ew, NGC0, NGC1, 0.0).reshape(NW, NGC0 * 128)
    ones_hbm = jnp.ones((128, 16), jnp.float32)

    degp = _deg_call(b_dst, c_dst, ones_hbm)
    ii2s = _stage1(ii, k_emb_w, w4_W, w4_b)
    hnbp = _segsum_call(ii2s, b_src, b_dst, None,
                        NGB0, NGB1)
    h_iI, scs = _stage3(hnbp, degp, e_emb, ek_Wself, ek_Wneigh, ek_b,
                        w2_W, w2_b, w1_W, w1_b)
    hncp = _segsum_call(scs, c_src, c_dst, ew,
                        NGC0, NGC1)
    return _stage5(hncp, degp, e_emb, h_iI, ee_Wself, ee_Wneigh, ee_b,
                   w3_W, w3_b, comb_W, comb_b)


# speedup vs baseline: 1.0301x; 1.0301x over previous
import functools

import jax
import jax.numpy as jnp
from jax import lax
from jax.experimental import pallas as pl
from jax.experimental.pallas import tpu as pltpu
from jax.experimental.pallas import tpu_sc as plsc

N_KN = 10000
N_EMP = 50000
EMB = 128

NC = 2
NS = 16
NW = NC * NS
NT = 4
CW = EMB // NT

TRASH = N_EMP
R_ACC = 50176
RPS = R_ACC // NS
ZCH = 112

NGB0, NGB1 = 64, 16
NGC0, NGC1 = 160, 80
NBUF = 4

_mesh = lambda: plsc.VectorSubcoreMesh(core_axis_name="c", subcore_axis_name="s")
_sc_params = lambda: pltpu.CompilerParams(use_tc_tiling_on_sc=False,
                                          needs_layout_passes=False)


def _prep_chunks(a, ng0, ng1, fill):
    n0, n1 = NS * ng0 * 128, NS * ng1 * 128
    pad = n0 + n1 - a.shape[0]
    if fill == TRASH:
        padv = TRASH + (jnp.arange(pad, dtype=a.dtype) % (R_ACC - N_EMP))
    else:
        padv = jnp.full((pad,), fill, a.dtype)
    a = jnp.concatenate([a, padv])
    ngm = max(ng0, ng1)
    p0 = a[:n0].reshape(NS, ng0, 128)
    p1 = a[n0:].reshape(NS, ng1, 128)
    p0 = jnp.pad(p0, ((0, 0), (0, ngm - ng0), (0, 0)), constant_values=fill)
    p1 = jnp.pad(p1, ((0, 0), (0, ngm - ng1), (0, 0)), constant_values=fill)
    return jnp.concatenate([p0, p1], axis=0)


def _zero_accum(acc, zbuf, s, zsem):
    @pl.loop(0, RPS // ZCH)
    def _(z):
        pltpu.async_copy(zbuf, acc.at[pl.ds(s * RPS + z * ZCH, ZCH)], zsem)

    @pl.loop(0, RPS // ZCH)
    def _(z):
        pltpu.make_async_copy(zbuf, acc.at[pl.ds(0, ZCH)], zsem).wait()


def _deg_call(dstb, dstc, ones_hbm):
    @functools.partial(
        pl.kernel,
        mesh=_mesh(),
        compiler_params=_sc_params(),
        out_type=jax.ShapeDtypeStruct((4, R_ACC, 16), jnp.float32),
        scratch_types=[
            pltpu.VMEM((NGB0, 128), jnp.int32),
            pltpu.VMEM((NGC0, 128), jnp.int32),
            pltpu.VMEM((128, 16), jnp.float32),
            pltpu.VMEM((ZCH, 16), jnp.float32),
            pltpu.VMEM_SHARED((R_ACC, 16), jnp.float32),
            pltpu.SemaphoreType.DMA,
            pltpu.SemaphoreType.DMA,
        ],
    )
    def k(dstb_hbm, dstc_hbm, ones_h, out_hbm, dbv, dcv, ones_v, zbuf, dacc,
          lsem, ssem):
        c = lax.axis_index("c")
        s = lax.axis_index("s")
        row = c * NS + s
        pltpu.sync_copy(ones_h, ones_v)

        @pl.loop(0, ZCH)
        def _(r):
            zbuf[r, pl.ds(0, 16)] = jnp.zeros((16,), jnp.float32)

        for set_i, (dh, dv, ng0, ng1) in enumerate(
                [(dstb_hbm, dbv, NGB0, NGB1), (dstc_hbm, dcv, NGC0, NGC1)]):
            ngc = jnp.where(c == 0, ng0, ng1)
            pltpu.sync_copy(dh.at[row], dv)
            _zero_accum(dacc, zbuf, s, lsem)
            plsc.subcore_barrier()

            @pl.loop(0, ngc)
            def _(g):
                pltpu.async_copy(ones_v, dacc.at[dv.at[g]], ssem, add=True)

                @pl.when(g >= 8)
                def _():
                    pltpu.make_async_copy(ones_h, ones_v, ssem).wait()

            @pl.loop(0, jnp.minimum(ngc, 8))
            def _(g):
                pltpu.make_async_copy(ones_h, ones_v, ssem).wait()

            plsc.subcore_barrier()
            pltpu.sync_copy(dacc.at[pl.ds(s * RPS, RPS)],
                            out_hbm.at[set_i * 2 + c, pl.ds(s * RPS, RPS)])
            plsc.subcore_barrier()

    return k(dstb, dstc, ones_hbm)


def _segsum_call(tables, srcc, dstc, ewc, ng0, ng1):
    weighted = ewc is not None
    ngm = max(ng0, ng1)

    scratch = (
        [pltpu.VMEM((NBUF, 128), jnp.int32) for _ in range(2)]
        + [pltpu.VMEM((NBUF, 128), jnp.int32) for _ in range(2)]
        + ([pltpu.VMEM((NBUF * 128,), jnp.float32) for _ in range(2)]
           if weighted else [])
        + [pltpu.VMEM((128, CW), jnp.float32) for _ in range(NBUF)]
        + [pltpu.VMEM((ZCH, CW), jnp.float32),
           pltpu.VMEM_SHARED((R_ACC, CW), jnp.float32)]
        + [pltpu.SemaphoreType.DMA for _ in range(2 * NBUF + 3)]
    )

    @functools.partial(
        pl.kernel,
        mesh=_mesh(),
        compiler_params=_sc_params(),
        out_type=jax.ShapeDtypeStruct((NC, NT, R_ACC, CW), jnp.float32),
        scratch_types=scratch,
    )
    def k(tab, src_hbm, dst_hbm, *rest):
        if weighted:
            (ew_hbm, out_hbm, sv0, sv1, dv0, dv1, ev0, ev1, *rest2) = rest
            evs = (ev0, ev1)
        else:
            (out_hbm, sv0, sv1, dv0, dv1, *rest2) = rest
        svs = (sv0, sv1)
        dvs = (dv0, dv1)
        bufs = rest2[:NBUF]
        zbuf = rest2[NBUF]
        acc = rest2[NBUF + 1]
        gsems = rest2[NBUF + 2:2 * NBUF + 2]
        ssems = rest2[2 * NBUF + 2:3 * NBUF + 2]
        isems = rest2[3 * NBUF + 2:3 * NBUF + 4]
        zsem = rest2[3 * NBUF + 4]
        c = lax.axis_index("c")
        s = lax.axis_index("s")
        wid = c * NS + s
        ng = jnp.where(c == 0, ng0, ng1)
        nch = ng // NBUF

        @pl.loop(0, ZCH)
        def _(r):
            zbuf[r, pl.ds(0, 16)] = jnp.zeros((16,), jnp.float32)
            zbuf[r, pl.ds(16, 16)] = jnp.zeros((16,), jnp.float32)

        def fetch_idx_sync(ci, p):
            pltpu.sync_copy(src_hbm.at[wid, pl.ds(ci * NBUF, NBUF)], svs[p])
            pltpu.sync_copy(dst_hbm.at[wid, pl.ds(ci * NBUF, NBUF)], dvs[p])
            if weighted:
                pltpu.sync_copy(
                    ew_hbm.at[wid, pl.ds(ci * NBUF * 128, NBUF * 128)],
                    evs[p])

        def fetch_idx_async(ci, p):
            pltpu.async_copy(src_hbm.at[wid, pl.ds(ci * NBUF, NBUF)],
                             svs[p], isems[p])
            pltpu.async_copy(dst_hbm.at[wid, pl.ds(ci * NBUF, NBUF)],
                             dvs[p], isems[p])
            if weighted:
                pltpu.async_copy(
                    ew_hbm.at[wid, pl.ds(ci * NBUF * 128, NBUF * 128)],
                    evs[p], isems[p])

        def wait_idx(p):
            pltpu.make_async_copy(src_hbm.at[0, pl.ds(0, NBUF)], svs[p],
                                  isems[p]).wait()
            pltpu.make_async_copy(dst_hbm.at[0, pl.ds(0, NBUF)], dvs[p],
                                  isems[p]).wait()
            if weighted:
                pltpu.make_async_copy(
                    ew_hbm.at[0, pl.ds(0, NBUF * 128)], evs[p],
                    isems[p]).wait()

        def gather(t, p, kk, b):
            pltpu.async_copy(tab.at[t].at[svs[p].at[kk]], bufs[b], gsems[b])

        def wait_gather(b):
            pltpu.make_async_copy(tab.at[0].at[pl.ds(0, 128)], bufs[b],
                                  gsems[b]).wait()

        def drain_scatter(b):
            pltpu.make_async_copy(tab.at[0].at[pl.ds(0, 128)], bufs[b],
                                  ssems[b]).wait()

        @pl.loop(0, NT)
        def _(t):
            _zero_accum(acc, zbuf, s, zsem)
            plsc.subcore_barrier()

            fetch_idx_sync(0, 0)
            gather(t, 0, 0, 0)
            gather(t, 0, 1, 1)

            @pl.loop(0, nch, step=2)
            def _(ci0):
                for p in range(2):
                    ci = ci0 + p
                    for kk in range(NBUF):
                        h = ci * NBUF + kk
                        b = kk
                        bn = (kk + 2) % NBUF
                        wait_gather(b)
                        if weighted:
                            rows = bufs[b]
                            ewv = evs[p]

                            @pl.loop(0, 128, step=8)
                            def _(e0):
                                for dd in range(8):
                                    e = e0 + dd
                                    w = plsc.load_gather(
                                        ewv,
                                        [jnp.full((16,), kk * 128,
                                                  jnp.int32) + e])
                                    rows[e, pl.ds(0, 16)] = (
                                        rows[e, pl.ds(0, 16)] * w)
                                    rows[e, pl.ds(16, 16)] = (
                                        rows[e, pl.ds(16, 16)] * w)

                        pltpu.async_copy(bufs[b], acc.at[dvs[p].at[kk]],
                                         ssems[b], add=True)

                        @pl.when(h >= 2)
                        def _():
                            drain_scatter(bn)

                        if kk == 1:
                            @pl.when(ci + 1 < nch)
                            def _():
                                fetch_idx_async(ci + 1, 1 - p)
                        if kk == 2:
                            @pl.when(ci + 1 < nch)
                            def _():
                                wait_idx(1 - p)

                        pn = p if kk < 2 else 1 - p
                        kn = (kk + 2) % NBUF

                        @pl.when(h + 2 < ng)
                        def _():
                            gather(t, pn, kn, bn)

            drain_scatter(2)
            drain_scatter(3)
            plsc.subcore_barrier()
            pltpu.sync_copy(acc.at[pl.ds(s * RPS, RPS)],
                            out_hbm.at[c, t, pl.ds(s * RPS, RPS)])
            plsc.subcore_barrier()

    args = (tables, srcc, dstc)
    if weighted:
        args = args + (ewc,)
    return k(*args)


def _dotT(x, w):
    return lax.dot_general(x, w, (((1,), (1,)), ((), ())),
                           preferred_element_type=jnp.float32)


_tc_params = lambda: pltpu.CompilerParams(
    dimension_semantics=("parallel",))


def _full(a):
    return pl.BlockSpec(a.shape, lambda i: (0,) * a.ndim)


def _stage1(ii, kw, w4_W, w4_b):
    br = 2000

    def body(ii_r, kw_r, w_r, b_r, o_r):
        y = (_dotT(ii_r[...], w_r[:, :EMB]) + _dotT(kw_r[...], w_r[:, EMB:])
             + b_r[...])
        for t in range(NT):
            o_r[t] = y[:, t * CW:(t + 1) * CW]

    return pl.pallas_call(
        body,
        compiler_params=_tc_params(),
        grid=(N_KN // br,),
        in_specs=[
            pl.BlockSpec((br, EMB), lambda i: (i, 0)),
            pl.BlockSpec((br, EMB), lambda i: (i, 0)),
            _full(w4_W),
            _full(w4_b),
        ],
        out_specs=pl.BlockSpec((NT, br, CW), lambda i: (0, i, 0)),
        out_shape=jax.ShapeDtypeStruct((NT, N_KN, CW), jnp.float32),
    )(ii, kw, w4_W, w4_b)


def _strip_mats(hp_r, deg, Wn0, Wn1):
    recip = 1.0 / jnp.maximum(deg, 1.0)
    hn0 = None
    hn1 = None
    for t in range(NT):
        st = (hp_r[0, t] + hp_r[1, t]) * recip
        c0 = _dotT(st, Wn0[:, t * CW:(t + 1) * CW])
        c1 = _dotT(st, Wn1[:, t * CW:(t + 1) * CW])
        hn0 = c0 if hn0 is None else hn0 + c0
        hn1 = c1 if hn1 is None else hn1 + c1
    return hn0, hn1


def _stage3(hnbp, degp, e_emb, ek_Wself, ek_Wneigh, ek_b, w2_W, w2_b,
            w1_W, w1_b):
    br = 2000

    def body(hp_r, dg_r, e_r, ws_r, wn_r, b_r, w2_r, w2b_r, w1_r, w1b_r,
             o_hi, o_sc):
        degb = dg_r[0, :, 0:1] + dg_r[1, :, 0:1]
        hn0, hn1 = _strip_mats(hp_r, degb, wn_r[0], wn_r[1])
        e = e_r[...]
        d1 = jnp.maximum(_dotT(e, ws_r[0]) + hn0 + b_r[0], 0.0)
        hi = _dotT(d1, ws_r[1]) + hn1 + b_r[1]
        hi2 = _dotT(hi, w2_r[...]) + w2b_r[...]
        o_hi[...] = hi2
        sc = (_dotT(hi2, w1_r[:, :EMB]) + _dotT(e, w1_r[:, EMB:])
              + w1b_r[...])
        for t in range(NT):
            o_sc[t] = sc[:, t * CW:(t + 1) * CW]

    return pl.pallas_call(
        body,
        compiler_params=_tc_params(),
        grid=(N_EMP // br,),
        in_specs=[
            pl.BlockSpec((NC, NT, br, CW), lambda i: (0, 0, i, 0)),
            pl.BlockSpec((4, br, 16), lambda i: (0, i, 0)),
            pl.BlockSpec((br, EMB), lambda i: (i, 0)),
            _full(ek_Wself), _full(ek_Wneigh), _full(ek_b),
            _full(w2_W), _full(w2_b), _full(w1_W), _full(w1_b),
        ],
        out_specs=[
            pl.BlockSpec((br, EMB), lambda i: (i, 0)),
            pl.BlockSpec((NT, br, CW), lambda i: (0, i, 0)),
        ],
        out_shape=[
            jax.ShapeDtypeStruct((N_EMP, EMB), jnp.float32),
            jax.ShapeDtypeStruct((NT, N_EMP, CW), jnp.float32),
        ],
    )(hnbp, degp, e_emb, ek_Wself, ek_Wneigh, ek_b, w2_W, w2_b, w1_W, w1_b)


def _stage5(hncp, degp, e_emb, h_iI, ee_Wself, ee_Wneigh, ee_b, w3_W, w3_b,
            comb_W, comb_b):
    br = 2000

    def body(hp_r, dg_r, e_r, hi_r, ws_r, wn_r, b_r, w3_r, w3b_r, cb_r,
             cbb_r, o_r):
        degc = dg_r[2, :, 0:1] + dg_r[3, :, 0:1]
        hn0, hn1 = _strip_mats(hp_r, degc, wn_r[0], wn_r[1])
        e = e_r[...]
        d2 = jnp.maximum(_dotT(e, ws_r[0]) + hn0 + b_r[0], 0.0)
        hs = _dotT(d2, ws_r[1]) + hn1 + b_r[1]
        hs2 = _dotT(hs, w3_r[...]) + w3b_r[...]
        h = (_dotT(hi_r[...], cb_r[:, :EMB]) + _dotT(hs2, cb_r[:, EMB:])
             + cbb_r[...])
        o_r[...] = jnp.where(h >= 0, h, 0.2 * h)

    return pl.pallas_call(
        body,
        compiler_params=_tc_params(),
        grid=(N_EMP // br,),
        in_specs=[
            pl.BlockSpec((NC, NT, br, CW), lambda i: (0, 0, i, 0)),
            pl.BlockSpec((4, br, 16), lambda i: (0, i, 0)),
            pl.BlockSpec((br, EMB), lambda i: (i, 0)),
            pl.BlockSpec((br, EMB), lambda i: (i, 0)),
            _full(ee_Wself), _full(ee_Wneigh), _full(ee_b),
            _full(w3_W), _full(w3_b), _full(comb_W), _full(comb_b),
        ],
        out_specs=pl.BlockSpec((br, EMB), lambda i: (i, 0)),
        out_shape=jax.ShapeDtypeStruct((N_EMP, EMB), jnp.float32),
    )(hncp, degp, e_emb, h_iI, ee_Wself, ee_Wneigh, ee_b, w3_W, w3_b,
      comb_W, comb_b)


def kernel(ii, e_emb, cf_ew, belong_edge_index, collab_edge_index, k_emb_w,
           w4_W, w4_b, w1_W, w1_b, w2_W, w2_b, w3_W, w3_b, comb_W, comb_b,
           ek_Wself, ek_Wneigh, ek_b, ee_Wself, ee_Wneigh, ee_b):
    b_src = _prep_chunks(belong_edge_index[0], NGB0, NGB1, 0)
    b_dst = _prep_chunks(belong_edge_index[1], NGB0, NGB1, TRASH)
    c_src = _prep_chunks(collab_edge_index[0], NGC0, NGC1, 0)
    c_dst = _prep_chunks(collab_edge_index[1], NGC0, NGC1, TRASH)
    ew = _prep_chunks(cf_ew, NGC0, NGC1, 0.0).reshape(NW, NGC0 * 128)
    ones_hbm = jnp.ones((128, 16), jnp.float32)

    degp = _deg_call(b_dst, c_dst, ones_hbm)
    ii2s = _stage1(ii, k_emb_w, w4_W, w4_b)
    hnbp = _segsum_call(ii2s, b_src, b_dst, None,
                        NGB0, NGB1)
    h_iI, scs = _stage3(hnbp, degp, e_emb, ek_Wself, ek_Wneigh, ek_b,
                        w2_W, w2_b, w1_W, w1_b)
    hncp = _segsum_call(scs, c_src, c_dst, ew,
                        NGC0, NGC1)
    return _stage5(hncp, degp, e_emb, h_iI, ee_Wself, ee_Wneigh, ee_b,
                   w3_W, w3_b, comb_W, comb_b)
